# Initial kernel scaffold; baseline (speedup 1.0000x reference)
import jax
import jax.numpy as jnp
from jax.experimental import pallas as pl


def kernel(atomic_numbers, distances, idx_i, idx_j, c6ab, rcov, r2r4, s6, s8, a1, a2):
    raise NotImplementedError("write your pallas kernel here")


# speedup vs baseline: 313.4688x; 313.4688x over previous
import functools

import jax
import jax.numpy as jnp
from jax import lax
from jax.experimental import pallas as pl
from jax.experimental.pallas import tpu as pltpu
from jax.experimental.pallas import tpu_sc as plsc

N = 50000
E = 1600000
ZMAX = 95
CUTOFF = 10.0
CUTON = 9.0
K1 = 16.0
K3 = -4.0
SQRT3 = 1.7320508075688772

NP = 50176
STRIPE = NP // 16
E2 = 1605632
EROWS = E2 // 128
TPW = EROWS // 32
DEAD = NP - 8

CA = 8
CB = 4
NCH1 = TPW // CA
NCH2 = TPW // CB

_mesh = plsc.VectorSubcoreMesh(core_axis_name="c", subcore_axis_name="s")


def _iota16():
    return lax.broadcasted_iota(jnp.int32, (16,), 0)


def _pass1_body(ii_h, ij_h, d_h, z_h, rc_h, sr_h,
                nc2_h, p_h, st_h,
                zbuf, rcb, srb, iib, ijb, ddb, dmpb, pb, stb, zrb, nc_sp):
    c = lax.axis_index("c")
    s = lax.axis_index("s")
    tid = c * 16 + s
    stripe = s * STRIPE

    pltpu.sync_copy(z_h, zbuf)
    pltpu.sync_copy(rc_h, rcb)
    pltpu.sync_copy(sr_h, srb)

    zeros16 = jnp.zeros((16,), jnp.float32)

    def zr(i, _):
        zrb[pl.ds(i * 16, 16)] = zeros16
        return 0
    lax.fori_loop(0, STRIPE // 16, zr, 0)
    pltpu.sync_copy(zrb, nc_sp.at[pl.ds(stripe, STRIPE)])
    plsc.subcore_barrier()

    def chunk(ch, _):
        rowbase = tid * TPW + ch * CA
        pltpu.sync_copy(ii_h.at[pl.ds(rowbase, CA)], iib)
        pltpu.sync_copy(ij_h.at[pl.ds(rowbase, CA)], ijb)
        pltpu.sync_copy(d_h.at[pl.ds(rowbase, CA)], ddb)

        def vblk(v, _):
            k = lax.shift_right_logical(v, 3)
            off = (v & 7) * 16
            sl = pl.ds(off, 16)
            ii_v = iib[k, sl]
            ij_v = ijb[k, sl]
            d_v = ddb[k, sl]
            zi = plsc.load_gather(zbuf, [ii_v])
            zj = plsc.load_gather(zbuf, [ij_v])
            rci = plsc.load_gather(rcb, [zi])
            rcj = plsc.load_gather(rcb, [zj])
            sri = plsc.load_gather(srb, [zi])
            srj = plsc.load_gather(srb, [zj])
            rr = (rci + rcj) / d_v
            damp = 1.0 / (1.0 + jnp.exp(K1 * (1.0 - rr)))
            x = CUTOFF - d_v
            poly = 1.0 + ((-6.0 * x + 15.0) * x - 10.0) * (x * x * x)
            damp = damp * jnp.where(x < CUTON, 1.0, poly)
            dmpb[k, sl] = damp
            pb[k, sl] = zi * ZMAX + zj
            stb[k, sl] = (SQRT3 * sri) * srj
            return 0
        lax.fori_loop(0, CA * 8, vblk, 0)

        pltpu.sync_copy(pb, p_h.at[pl.ds(rowbase, CA)])
        pltpu.sync_copy(stb, st_h.at[pl.ds(rowbase, CA)])
        for k in range(CA):
            pltpu.sync_copy(dmpb.at[k], nc_sp.at[iib.at[k]], add=True)
        return 0
    lax.fori_loop(0, NCH1, chunk, 0)

    plsc.subcore_barrier()
    pltpu.sync_copy(nc_sp.at[pl.ds(stripe, STRIPE)], zrb)
    pltpu.sync_copy(zrb, nc2_h.at[pl.ds(c * NP + stripe, STRIPE)])


def _pass2_body(ii_h, ij_h, d_h, p_h, st_h, nc2_h, c6t_h, par_h,
                out2_h,
                ncb, b0, b1, iib, ijb, ddb, pbb, stbb, rows, ebuf, parb,
                sem, nc_sp, out_sp):
    c = lax.axis_index("c")
    s = lax.axis_index("s")
    tid = c * 16 + s
    stripe = s * STRIPE
    ssl = pl.ds(stripe, STRIPE)

    pltpu.sync_copy(par_h, parb)
    pltpu.sync_copy(nc2_h.at[pl.ds(stripe, STRIPE)], b0)
    pltpu.sync_copy(nc2_h.at[pl.ds(NP + stripe, STRIPE)], b1)

    def mrg(i, _):
        sl = pl.ds(i * 16, 16)
        b0[sl] = b0[sl] + b1[sl]
        return 0
    lax.fori_loop(0, STRIPE // 16, mrg, 0)
    pltpu.sync_copy(b0, nc_sp.at[ssl])

    zeros16 = jnp.zeros((16,), jnp.float32)

    def zr(i, _):
        b1[pl.ds(i * 16, 16)] = zeros16
        return 0
    lax.fori_loop(0, STRIPE // 16, zr, 0)
    pltpu.sync_copy(b1, out_sp.at[ssl])
    plsc.subcore_barrier()
    pltpu.sync_copy(nc_sp, ncb)

    s6v = parb[0, :16]
    s8v = parb[1, :16]
    a1v = parb[2, :16]
    a2v = parb[3, :16]
    cut6 = CUTOFF ** 6
    cut8 = CUTOFF ** 8
    iota = _iota16()

    def chunk(ch, _):
        rowbase = tid * TPW + ch * CB
        pltpu.sync_copy(ii_h.at[pl.ds(rowbase, CB)], iib)
        pltpu.sync_copy(ij_h.at[pl.ds(rowbase, CB)], ijb)
        pltpu.sync_copy(d_h.at[pl.ds(rowbase, CB)], ddb)
        pltpu.sync_copy(p_h.at[pl.ds(rowbase, CB)], pbb)
        pltpu.sync_copy(st_h.at[pl.ds(rowbase, CB)], stbb)
        cps = [pltpu.async_copy(c6t_h.at[pbb.at[k]],
                                rows.at[pl.ds(k * 128, 128)], sem)
               for k in range(CB)]
        for cp in cps:
            cp.wait()

        def vblk(v, _):
            k = lax.shift_right_logical(v, 3)
            off = (v & 7) * 16
            sl = pl.ds(off, 16)
            ii_v = iib[k, sl]
            ij_v = ijb[k, sl]
            d_v = ddb[k, sl]
            st_v = stbb[k, sl]
            nci = plsc.load_gather(ncb, [ii_v])
            ncj = plsc.load_gather(ncb, [ij_v])
            e_v = iota + v * 16
            rsum = jnp.zeros((16,), jnp.float32)
            csum = jnp.zeros((16,), jnp.float32)
            for cell in range(25):
                col = cell * 3
                cn0 = plsc.load_gather(rows, [e_v, jnp.full((16,), col, jnp.int32)])
                cn1 = plsc.load_gather(rows, [e_v, jnp.full((16,), col + 1, jnp.int32)])
                cn2 = plsc.load_gather(rows, [e_v, jnp.full((16,), col + 2, jnp.int32)])
                dx = cn1 - nci
                dy = cn2 - ncj
                t = jnp.exp(K3 * (dx * dx + dy * dy))
                m = cn0 > 0.0
                rsum = rsum + jnp.where(m, t, 0.0)
                csum = csum + jnp.where(m, t * cn0, 0.0)
            c6 = jnp.where(rsum > 0.0, csum / rsum, -1e30)
            c8 = (c6 * st_v) * st_v
            tmp = a1v * st_v + a2v
            tmp2 = tmp * tmp
            tmp6 = tmp2 * tmp2 * tmp2
            tmp8 = tmp6 * tmp2
            d2 = d_v * d_v
            d6 = d2 * d2 * d2
            d8 = d6 * d2
            c6t6 = cut6 + tmp6
            c8t8 = cut8 + tmp8
            dd = d_v * (1.0 / CUTOFF) - 1.0
            e6 = 1.0 / (d6 + tmp6) - 1.0 / c6t6 + (6.0 * cut6) / (c6t6 * c6t6) * dd
            e8 = 1.0 / (d8 + tmp8) - 1.0 / c8t8 + (8.0 * cut8) / (c8t8 * c8t8) * dd
            inside = d_v < CUTOFF
            e6 = jnp.where(inside, e6, 0.0)
            e8 = jnp.where(inside, e8, 0.0)
            e6 = (-0.5 * s6v) * c6 * e6
            e8 = (-0.5 * s8v) * c8 * e8
            ebuf[k, sl] = e6 + e8
            return 0
        lax.fori_loop(0, CB * 8, vblk, 0)

        for k in range(CB):
            pltpu.sync_copy(ebuf.at[k], out_sp.at[iib.at[k]], add=True)
        return 0
    lax.fori_loop(0, NCH2, chunk, 0)

    plsc.subcore_barrier()
    pltpu.sync_copy(out_sp.at[ssl], b0)
    pltpu.sync_copy(b0, out2_h.at[pl.ds(c * NP + stripe, STRIPE)])


def _merge_body(a_ref, o_ref):
    o_ref[...] = a_ref[0] + a_ref[1]


def kernel(atomic_numbers, distances, idx_i, idx_j, c6ab, rcov, r2r4,
           s6, s8, a1, a2):
    f32 = jnp.float32
    i32 = jnp.int32
    zp = jnp.concatenate([atomic_numbers.astype(i32),
                          jnp.zeros((NP - N,), i32)])
    pad = E2 - E
    ii2 = jnp.concatenate([idx_i.astype(i32),
                           jnp.full((pad,), DEAD, i32)]).reshape(EROWS, 128)
    ij2 = jnp.concatenate([idx_j.astype(i32),
                           jnp.full((pad,), DEAD, i32)]).reshape(EROWS, 128)
    d2 = jnp.concatenate([distances,
                          jnp.full((pad,), 11.0, f32)]).reshape(EROWS, 128)
    rc96 = jnp.concatenate([rcov, jnp.ones((1,), f32)])
    sr96 = jnp.concatenate([jnp.sqrt(r2r4), jnp.ones((1,), f32)])
    c6t = c6ab.reshape(ZMAX * ZMAX, 75)
    par = jnp.stack([jnp.broadcast_to(s6, (16,)), jnp.broadcast_to(s8, (16,)),
                     jnp.broadcast_to(a1, (16,)), jnp.broadcast_to(a2, (16,))])

    pass1 = pl.kernel(
        _pass1_body,
        out_type=(jax.ShapeDtypeStruct((2 * NP,), f32),
                  jax.ShapeDtypeStruct((EROWS, 128), i32),
                  jax.ShapeDtypeStruct((EROWS, 128), f32)),
        mesh=_mesh,
        scratch_types=[
            pltpu.VMEM((NP,), i32),
            pltpu.VMEM((96,), f32),
            pltpu.VMEM((96,), f32),
            pltpu.VMEM((CA, 128), i32),
            pltpu.VMEM((CA, 128), i32),
            pltpu.VMEM((CA, 128), f32),
            pltpu.VMEM((CA, 128), f32),
            pltpu.VMEM((CA, 128), i32),
            pltpu.VMEM((CA, 128), f32),
            pltpu.VMEM((STRIPE,), f32),
            pltpu.VMEM_SHARED((NP,), f32),
        ],
        compiler_params=pltpu.CompilerParams(needs_layout_passes=False, use_tc_tiling_on_sc=False),
        name="d3_pass1",
    )
    nc2, p2, st2 = pass1(ii2, ij2, d2, zp, rc96, sr96)

    pass2 = pl.kernel(
        _pass2_body,
        out_type=jax.ShapeDtypeStruct((2 * NP,), f32),
        mesh=_mesh,
        scratch_types=[
            pltpu.VMEM((NP,), f32),
            pltpu.VMEM((STRIPE,), f32),
            pltpu.VMEM((STRIPE,), f32),
            pltpu.VMEM((CB, 128), i32),
            pltpu.VMEM((CB, 128), i32),
            pltpu.VMEM((CB, 128), f32),
            pltpu.VMEM((CB, 128), i32),
            pltpu.VMEM((CB, 128), f32),
            pltpu.VMEM((CB * 128, 75), f32),
            pltpu.VMEM((CB, 128), f32),
            pltpu.VMEM((4, 16), f32),
            pltpu.SemaphoreType.DMA,
            pltpu.VMEM_SHARED((NP,), f32),
            pltpu.VMEM_SHARED((NP,), f32),
        ],
        compiler_params=pltpu.CompilerParams(needs_layout_passes=False, use_tc_tiling_on_sc=False),
        name="d3_pass2",
    )
    out2 = pass2(ii2, ij2, d2, p2, st2, nc2, c6t, par)

    merged = pl.pallas_call(
        _merge_body,
        out_shape=jax.ShapeDtypeStruct((EROWS // 32, 128), f32),
        grid=(EROWS // 32 // 56,),
        in_specs=[pl.BlockSpec((2, 56, 128), lambda i: (0, i, 0))],
        out_specs=pl.BlockSpec((56, 128), lambda i: (i, 0)),
    )(out2.reshape(2, EROWS // 32, 128))
    return merged.reshape(NP)[:N]


# speedup vs baseline: 412.8824x; 1.3171x over previous
import functools

import jax
import jax.numpy as jnp
from jax import lax
from jax.experimental import pallas as pl
from jax.experimental.pallas import tpu as pltpu
from jax.experimental.pallas import tpu_sc as plsc

N = 50000
E = 1600000
ZMAX = 95
CUTOFF = 10.0
CUTON = 9.0
K1 = 16.0
K3 = -4.0
SQRT3 = 1.7320508075688772

NP = 50176
STRIPE = NP // 16
E2 = 1605632
EROWS = E2 // 128
TPW = EROWS // 32
DEAD = NP - 8

CA = 8
CB = 2
NCH1 = TPW // CA
NCH2 = TPW // CB

_mesh = plsc.VectorSubcoreMesh(core_axis_name="c", subcore_axis_name="s")


def _iota16():
    return lax.broadcasted_iota(jnp.int32, (16,), 0)


def _pass1_body(ii_h, ij_h, d_h, z_h, rc_h, sr_h,
                nc2_h, p_h, st_h,
                zbuf, rcb, srb, iib, ijb, ddb, dmpb, pb, stb, zrb, nc_sp):
    c = lax.axis_index("c")
    s = lax.axis_index("s")
    tid = c * 16 + s
    stripe = s * STRIPE

    pltpu.sync_copy(z_h, zbuf)
    pltpu.sync_copy(rc_h, rcb)
    pltpu.sync_copy(sr_h, srb)

    zeros16 = jnp.zeros((16,), jnp.float32)

    def zr(i, _):
        zrb[pl.ds(i * 16, 16)] = zeros16
        return 0
    lax.fori_loop(0, STRIPE // 16, zr, 0)
    pltpu.sync_copy(zrb, nc_sp.at[pl.ds(stripe, STRIPE)])
    plsc.subcore_barrier()

    def chunk(ch, _):
        rowbase = tid * TPW + ch * CA
        pltpu.sync_copy(ii_h.at[pl.ds(rowbase, CA)], iib)
        pltpu.sync_copy(ij_h.at[pl.ds(rowbase, CA)], ijb)
        pltpu.sync_copy(d_h.at[pl.ds(rowbase, CA)], ddb)

        def vblk(v, _):
            k = lax.shift_right_logical(v, 3)
            off = (v & 7) * 16
            sl = pl.ds(off, 16)
            ii_v = iib[k, sl]
            ij_v = ijb[k, sl]
            d_v = ddb[k, sl]
            zi = plsc.load_gather(zbuf, [ii_v])
            zj = plsc.load_gather(zbuf, [ij_v])
            rci = plsc.load_gather(rcb, [zi])
            rcj = plsc.load_gather(rcb, [zj])
            sri = plsc.load_gather(srb, [zi])
            srj = plsc.load_gather(srb, [zj])
            rr = (rci + rcj) / d_v
            damp = 1.0 / (1.0 + jnp.exp(K1 * (1.0 - rr)))
            x = CUTOFF - d_v
            poly = 1.0 + ((-6.0 * x + 15.0) * x - 10.0) * (x * x * x)
            damp = damp * jnp.where(x < CUTON, 1.0, poly)
            dmpb[k, sl] = damp
            pb[k, sl] = zi * ZMAX + zj
            stb[k, sl] = (SQRT3 * sri) * srj
            return 0
        lax.fori_loop(0, CA * 8, vblk, 0)

        pltpu.sync_copy(pb, p_h.at[pl.ds(rowbase, CA)])
        pltpu.sync_copy(stb, st_h.at[pl.ds(rowbase, CA)])
        for k in range(CA):
            pltpu.sync_copy(dmpb.at[k], nc_sp.at[iib.at[k]], add=True)
        return 0
    lax.fori_loop(0, NCH1, chunk, 0)

    plsc.subcore_barrier()
    pltpu.sync_copy(nc_sp.at[pl.ds(stripe, STRIPE)], zrb)
    pltpu.sync_copy(zrb, nc2_h.at[pl.ds(c * NP + stripe, STRIPE)])


def _pass2_body(ii_h, ij_h, d_h, p_h, st_h, nc2_h, c6t_h, par_h,
                out2_h,
                ncb, b0, b1, iib, ijb, ddb, pbb, stbb, rows, ebuf, parb,
                sem_l0, sem_l1, sem_i0, sem_i1, nc_sp, out_sp):
    c = lax.axis_index("c")
    s = lax.axis_index("s")
    tid = c * 16 + s
    stripe = s * STRIPE
    ssl = pl.ds(stripe, STRIPE)
    sem_l = (sem_l0, sem_l1)
    sem_i = (sem_i0, sem_i1)

    pltpu.sync_copy(par_h, parb)
    pltpu.sync_copy(nc2_h.at[pl.ds(stripe, STRIPE)], b0)
    pltpu.sync_copy(nc2_h.at[pl.ds(NP + stripe, STRIPE)], b1)

    def mrg(i, _):
        sl = pl.ds(i * 16, 16)
        b0[sl] = b0[sl] + b1[sl]
        return 0
    lax.fori_loop(0, STRIPE // 16, mrg, 0)
    pltpu.sync_copy(b0, nc_sp.at[ssl])

    zeros16 = jnp.zeros((16,), jnp.float32)

    def zr(i, _):
        b1[pl.ds(i * 16, 16)] = zeros16
        return 0
    lax.fori_loop(0, STRIPE // 16, zr, 0)
    pltpu.sync_copy(b1, out_sp.at[ssl])
    plsc.subcore_barrier()
    pltpu.sync_copy(nc_sp, ncb)

    s6v = parb[0, :16]
    s8v = parb[1, :16]
    a1v = parb[2, :16]
    a2v = parb[3, :16]
    cut6 = CUTOFF ** 6
    cut8 = CUTOFF ** 8
    iota = _iota16()

    def lin_copies(idx, b):
        rowbase = tid * TPW + idx * CB
        rsl = pl.ds(rowbase, CB)
        return [(ii_h.at[rsl], iib.at[b]), (ij_h.at[rsl], ijb.at[b]),
                (d_h.at[rsl], ddb.at[b]), (p_h.at[rsl], pbb.at[b]),
                (st_h.at[rsl], stbb.at[b])]

    def issue_linear(idx, b):
        for src, dst in lin_copies(idx, b):
            pltpu.async_copy(src, dst, sem_l[b])

    def wait_linear(idx, b):
        for src, dst in lin_copies(idx, b):
            pltpu.make_async_copy(src, dst, sem_l[b]).wait()

    def ind_copies(idx, b):
        return [(c6t_h.at[pbb.at[b, k]], rows.at[b, pl.ds(k * 128, 128)])
                for k in range(CB)]

    def issue_indirect(idx, b):
        for src, dst in ind_copies(idx, b):
            pltpu.async_copy(src, dst, sem_i[b])

    def wait_indirect(idx, b):
        for src, dst in ind_copies(idx, b):
            pltpu.make_async_copy(src, dst, sem_i[b]).wait()

    def compute(idx, b):
        rref = rows.at[b]

        def vblk(v, _):
            k = lax.shift_right_logical(v, 3)
            off = (v & 7) * 16
            sl = pl.ds(off, 16)
            ii_v = iib[b, k, sl]
            ij_v = ijb[b, k, sl]
            d_v = ddb[b, k, sl]
            st_v = stbb[b, k, sl]
            nci = plsc.load_gather(ncb, [ii_v])
            ncj = plsc.load_gather(ncb, [ij_v])
            e_v = iota + v * 16
            rsum = jnp.zeros((16,), jnp.float32)
            csum = jnp.zeros((16,), jnp.float32)
            for cell in range(25):
                col = cell * 3
                cn0 = plsc.load_gather(rref, [e_v, jnp.full((16,), col, jnp.int32)])
                cn1 = plsc.load_gather(rref, [e_v, jnp.full((16,), col + 1, jnp.int32)])
                cn2 = plsc.load_gather(rref, [e_v, jnp.full((16,), col + 2, jnp.int32)])
                dx = cn1 - nci
                dy = cn2 - ncj
                t = jnp.exp(K3 * (dx * dx + dy * dy))
                m = cn0 > 0.0
                rsum = rsum + jnp.where(m, t, 0.0)
                csum = csum + jnp.where(m, t * cn0, 0.0)
            c6 = jnp.where(rsum > 0.0, csum / rsum, -1e30)
            c8 = (c6 * st_v) * st_v
            tmp = a1v * st_v + a2v
            tmp2 = tmp * tmp
            tmp6 = tmp2 * tmp2 * tmp2
            tmp8 = tmp6 * tmp2
            d2 = d_v * d_v
            d6 = d2 * d2 * d2
            d8 = d6 * d2
            c6t6 = cut6 + tmp6
            c8t8 = cut8 + tmp8
            dd = d_v * (1.0 / CUTOFF) - 1.0
            e6 = 1.0 / (d6 + tmp6) - 1.0 / c6t6 + (6.0 * cut6) / (c6t6 * c6t6) * dd
            e8 = 1.0 / (d8 + tmp8) - 1.0 / c8t8 + (8.0 * cut8) / (c8t8 * c8t8) * dd
            inside = d_v < CUTOFF
            e6 = jnp.where(inside, e6, 0.0)
            e8 = jnp.where(inside, e8, 0.0)
            e6 = (-0.5 * s6v) * c6 * e6
            e8 = (-0.5 * s8v) * c8 * e8
            ebuf[k, sl] = e6 + e8
            return 0
        lax.fori_loop(0, CB * 8, vblk, 0)

        for k in range(CB):
            pltpu.sync_copy(ebuf.at[k], out_sp.at[iib.at[b, k]], add=True)

    issue_linear(0, 0)
    wait_linear(0, 0)
    issue_indirect(0, 0)
    issue_linear(1, 1)

    def pair(g, _):
        for b in (0, 1):
            idx = g * 2 + b
            other = 1 - b

            @pl.when(idx + 1 < NCH2)
            def _():
                wait_linear(idx + 1, other)
                issue_indirect(idx + 1, other)

            wait_indirect(idx, b)
            compute(idx, b)

            @pl.when(idx + 2 < NCH2)
            def _():
                issue_linear(idx + 2, b)
        return 0
    lax.fori_loop(0, NCH2 // 2, pair, 0)

    plsc.subcore_barrier()
    pltpu.sync_copy(out_sp.at[ssl], b0)
    pltpu.sync_copy(b0, out2_h.at[pl.ds(c * NP + stripe, STRIPE)])


def _merge_body(a_ref, o_ref):
    o_ref[...] = a_ref[0] + a_ref[1]


def kernel(atomic_numbers, distances, idx_i, idx_j, c6ab, rcov, r2r4,
           s6, s8, a1, a2):
    f32 = jnp.float32
    i32 = jnp.int32
    zp = jnp.concatenate([atomic_numbers.astype(i32),
                          jnp.zeros((NP - N,), i32)])
    pad = E2 - E
    ii2 = jnp.concatenate([idx_i.astype(i32),
                           jnp.full((pad,), DEAD, i32)]).reshape(EROWS, 128)
    ij2 = jnp.concatenate([idx_j.astype(i32),
                           jnp.full((pad,), DEAD, i32)]).reshape(EROWS, 128)
    d2 = jnp.concatenate([distances,
                          jnp.full((pad,), 11.0, f32)]).reshape(EROWS, 128)
    rc96 = jnp.concatenate([rcov, jnp.ones((1,), f32)])
    sr96 = jnp.concatenate([jnp.sqrt(r2r4), jnp.ones((1,), f32)])
    c6t = c6ab.reshape(ZMAX * ZMAX, 75)
    par = jnp.stack([jnp.broadcast_to(s6, (16,)), jnp.broadcast_to(s8, (16,)),
                     jnp.broadcast_to(a1, (16,)), jnp.broadcast_to(a2, (16,))])

    pass1 = pl.kernel(
        _pass1_body,
        out_type=(jax.ShapeDtypeStruct((2 * NP,), f32),
                  jax.ShapeDtypeStruct((EROWS, 128), i32),
                  jax.ShapeDtypeStruct((EROWS, 128), f32)),
        mesh=_mesh,
        scratch_types=[
            pltpu.VMEM((NP,), i32),
            pltpu.VMEM((96,), f32),
            pltpu.VMEM((96,), f32),
            pltpu.VMEM((CA, 128), i32),
            pltpu.VMEM((CA, 128), i32),
            pltpu.VMEM((CA, 128), f32),
            pltpu.VMEM((CA, 128), f32),
            pltpu.VMEM((CA, 128), i32),
            pltpu.VMEM((CA, 128), f32),
            pltpu.VMEM((STRIPE,), f32),
            pltpu.VMEM_SHARED((NP,), f32),
        ],
        compiler_params=pltpu.CompilerParams(needs_layout_passes=False, use_tc_tiling_on_sc=False),
        name="d3_pass1",
    )
    nc2, p2, st2 = pass1(ii2, ij2, d2, zp, rc96, sr96)

    pass2 = pl.kernel(
        _pass2_body,
        out_type=jax.ShapeDtypeStruct((2 * NP,), f32),
        mesh=_mesh,
        scratch_types=[
            pltpu.VMEM((NP,), f32),
            pltpu.VMEM((STRIPE,), f32),
            pltpu.VMEM((STRIPE,), f32),
            pltpu.VMEM((2, CB, 128), i32),
            pltpu.VMEM((2, CB, 128), i32),
            pltpu.VMEM((2, CB, 128), f32),
            pltpu.VMEM((2, CB, 128), i32),
            pltpu.VMEM((2, CB, 128), f32),
            pltpu.VMEM((2, CB * 128, 75), f32),
            pltpu.VMEM((CB, 128), f32),
            pltpu.VMEM((4, 16), f32),
            pltpu.SemaphoreType.DMA,
            pltpu.SemaphoreType.DMA,
            pltpu.SemaphoreType.DMA,
            pltpu.SemaphoreType.DMA,
            pltpu.VMEM_SHARED((NP,), f32),
            pltpu.VMEM_SHARED((NP,), f32),
        ],
        compiler_params=pltpu.CompilerParams(needs_layout_passes=False, use_tc_tiling_on_sc=False),
        name="d3_pass2",
    )
    out2 = pass2(ii2, ij2, d2, p2, st2, nc2, c6t, par)

    merged = pl.pallas_call(
        _merge_body,
        out_shape=jax.ShapeDtypeStruct((EROWS // 32, 128), f32),
        grid=(EROWS // 32 // 56,),
        in_specs=[pl.BlockSpec((2, 56, 128), lambda i: (0, i, 0))],
        out_specs=pl.BlockSpec((56, 128), lambda i: (i, 0)),
    )(out2.reshape(2, EROWS // 32, 128))
    return merged.reshape(NP)[:N]


# speedup vs baseline: 443.4777x; 1.0741x over previous
import functools

import jax
import jax.numpy as jnp
from jax import lax
from jax.experimental import pallas as pl
from jax.experimental.pallas import tpu as pltpu
from jax.experimental.pallas import tpu_sc as plsc

N = 50000
E = 1600000
ZMAX = 95
CUTOFF = 10.0
CUTON = 9.0
K1 = 16.0
K3 = -4.0
SQRT3 = 1.7320508075688772

NP = 50176
STRIPE = NP // 16
E2 = 1605632
EROWS = E2 // 128
TPW = EROWS // 32
DEAD = NP - 8

CA = 7
CB = 2
NCH1 = TPW // CA
NCH2 = TPW // CB

_mesh = plsc.VectorSubcoreMesh(core_axis_name="c", subcore_axis_name="s")


def _iota16():
    return lax.broadcasted_iota(jnp.int32, (16,), 0)


def _pass1_body(ii_h, ij_h, d_h, z_h, rc_h, sr_h,
                nc2_h, p_h, st_h,
                zbuf, rcb, srb, iib, ijb, ddb, dmpb, pb, stb, zrb,
                sem_l0, sem_l1, sem_o0, sem_o1, nc_sp):
    c = lax.axis_index("c")
    s = lax.axis_index("s")
    tid = c * 16 + s
    stripe = s * STRIPE
    sem_l = (sem_l0, sem_l1)
    sem_o = (sem_o0, sem_o1)

    pltpu.sync_copy(z_h, zbuf)
    pltpu.sync_copy(rc_h, rcb)
    pltpu.sync_copy(sr_h, srb)

    zeros16 = jnp.zeros((16,), jnp.float32)

    def zr(i, _):
        zrb[pl.ds(i * 16, 16)] = zeros16
        return 0
    lax.fori_loop(0, STRIPE // 16, zr, 0)
    pltpu.sync_copy(zrb, nc_sp.at[pl.ds(stripe, STRIPE)])
    plsc.subcore_barrier()

    def lin_copies(idx, b):
        rsl = pl.ds(tid * TPW + idx * CA, CA)
        return [(ii_h.at[rsl], iib.at[b]), (ij_h.at[rsl], ijb.at[b]),
                (d_h.at[rsl], ddb.at[b])]

    def out_copies(idx, b):
        rsl = pl.ds(tid * TPW + idx * CA, CA)
        return [(pb.at[b], p_h.at[rsl]), (stb.at[b], st_h.at[rsl])]

    def issue(copies, sem):
        for src, dst in copies:
            pltpu.async_copy(src, dst, sem)

    def drain(copies, sem):
        for src, dst in copies:
            pltpu.make_async_copy(src, dst, sem).wait()

    def chunk(idx, b):
        drain(lin_copies(idx, b), sem_l[b])

        @pl.when(idx >= 2)
        def _():
            drain(out_copies(idx - 2, b), sem_o[b])

        def vblk(v, _):
            k = lax.shift_right_logical(v, 3)
            off = (v & 7) * 16
            sl = pl.ds(off, 16)
            ii_v = iib[b, k, sl]
            ij_v = ijb[b, k, sl]
            d_v = ddb[b, k, sl]
            zi = plsc.load_gather(zbuf, [ii_v])
            zj = plsc.load_gather(zbuf, [ij_v])
            rci = plsc.load_gather(rcb, [zi])
            rcj = plsc.load_gather(rcb, [zj])
            sri = plsc.load_gather(srb, [zi])
            srj = plsc.load_gather(srb, [zj])
            rr = (rci + rcj) / d_v
            damp = 1.0 / (1.0 + jnp.exp(K1 * (1.0 - rr)))
            x = CUTOFF - d_v
            poly = 1.0 + ((-6.0 * x + 15.0) * x - 10.0) * (x * x * x)
            damp = damp * jnp.where(x < CUTON, 1.0, poly)
            dmpb[b, k, sl] = damp
            pb[b, k, sl] = zi * ZMAX + zj
            stb[b, k, sl] = (SQRT3 * sri) * srj
            return 0
        lax.fori_loop(0, CA * 8, vblk, 0)

        for k in range(CA):
            pltpu.sync_copy(dmpb.at[b, k], nc_sp.at[iib.at[b, k]], add=True)
        issue(out_copies(idx, b), sem_o[b])

        @pl.when(idx + 2 < NCH1)
        def _():
            issue(lin_copies(idx + 2, b), sem_l[b])

    issue(lin_copies(0, 0), sem_l[0])
    issue(lin_copies(1, 1), sem_l[1])

    def pair(g, _):
        for b in (0, 1):
            chunk(g * 2 + b, b)
        return 0
    lax.fori_loop(0, NCH1 // 2, pair, 0)

    drain(out_copies(NCH1 - 2, 0), sem_o[0])
    drain(out_copies(NCH1 - 1, 1), sem_o[1])

    plsc.subcore_barrier()
    pltpu.sync_copy(nc_sp.at[pl.ds(stripe, STRIPE)], zrb)
    pltpu.sync_copy(zrb, nc2_h.at[pl.ds(c * NP + stripe, STRIPE)])


def _pass2_body(ii_h, ij_h, d_h, p_h, st_h, nc2_h, c6t_h, par_h,
                out2_h,
                ncb, b0, b1, iib, ijb, ddb, pbb, stbb, rows, ebuf, parb,
                sem_l0, sem_l1, sem_i0, sem_i1, nc_sp, out_sp):
    c = lax.axis_index("c")
    s = lax.axis_index("s")
    tid = c * 16 + s
    stripe = s * STRIPE
    ssl = pl.ds(stripe, STRIPE)
    sem_l = (sem_l0, sem_l1)
    sem_i = (sem_i0, sem_i1)

    pltpu.sync_copy(par_h, parb)
    pltpu.sync_copy(nc2_h.at[pl.ds(stripe, STRIPE)], b0)
    pltpu.sync_copy(nc2_h.at[pl.ds(NP + stripe, STRIPE)], b1)

    def mrg(i, _):
        sl = pl.ds(i * 16, 16)
        b0[sl] = b0[sl] + b1[sl]
        return 0
    lax.fori_loop(0, STRIPE // 16, mrg, 0)
    pltpu.sync_copy(b0, nc_sp.at[ssl])

    zeros16 = jnp.zeros((16,), jnp.float32)

    def zr(i, _):
        b1[pl.ds(i * 16, 16)] = zeros16
        return 0
    lax.fori_loop(0, STRIPE // 16, zr, 0)
    pltpu.sync_copy(b1, out_sp.at[ssl])
    plsc.subcore_barrier()
    pltpu.sync_copy(nc_sp, ncb)

    s6v = parb[0, :16]
    s8v = parb[1, :16]
    a1v = parb[2, :16]
    a2v = parb[3, :16]
    cut6 = CUTOFF ** 6
    cut8 = CUTOFF ** 8
    iota = _iota16()

    def lin_copies(idx, b):
        rowbase = tid * TPW + idx * CB
        rsl = pl.ds(rowbase, CB)
        return [(ii_h.at[rsl], iib.at[b]), (ij_h.at[rsl], ijb.at[b]),
                (d_h.at[rsl], ddb.at[b]), (p_h.at[rsl], pbb.at[b]),
                (st_h.at[rsl], stbb.at[b])]

    def issue_linear(idx, b):
        for src, dst in lin_copies(idx, b):
            pltpu.async_copy(src, dst, sem_l[b])

    def wait_linear(idx, b):
        for src, dst in lin_copies(idx, b):
            pltpu.make_async_copy(src, dst, sem_l[b]).wait()

    def ind_copies(idx, b):
        return [(c6t_h.at[pbb.at[b, k]], rows.at[b, pl.ds(k * 128, 128)])
                for k in range(CB)]

    def issue_indirect(idx, b):
        for src, dst in ind_copies(idx, b):
            pltpu.async_copy(src, dst, sem_i[b])

    def wait_indirect(idx, b):
        for src, dst in ind_copies(idx, b):
            pltpu.make_async_copy(src, dst, sem_i[b]).wait()

    def compute(idx, b):
        rref = rows.at[b]

        def vblk(v, _):
            k = lax.shift_right_logical(v, 3)
            off = (v & 7) * 16
            sl = pl.ds(off, 16)
            ii_v = iib[b, k, sl]
            ij_v = ijb[b, k, sl]
            d_v = ddb[b, k, sl]
            st_v = stbb[b, k, sl]
            nci = plsc.load_gather(ncb, [ii_v])
            ncj = plsc.load_gather(ncb, [ij_v])
            e_v = iota + v * 16
            rsum = jnp.zeros((16,), jnp.float32)
            csum = jnp.zeros((16,), jnp.float32)
            for cell in range(25):
                col = cell * 3
                cn0 = plsc.load_gather(rref, [e_v, jnp.full((16,), col, jnp.int32)])
                cn1 = plsc.load_gather(rref, [e_v, jnp.full((16,), col + 1, jnp.int32)])
                cn2 = plsc.load_gather(rref, [e_v, jnp.full((16,), col + 2, jnp.int32)])
                dx = cn1 - nci
                dy = cn2 - ncj
                t = jnp.exp(K3 * (dx * dx + dy * dy))
                m = cn0 > 0.0
                rsum = rsum + jnp.where(m, t, 0.0)
                csum = csum + jnp.where(m, t * cn0, 0.0)
            c6 = jnp.where(rsum > 0.0, csum / rsum, -1e30)
            c8 = (c6 * st_v) * st_v
            tmp = a1v * st_v + a2v
            tmp2 = tmp * tmp
            tmp6 = tmp2 * tmp2 * tmp2
            tmp8 = tmp6 * tmp2
            d2 = d_v * d_v
            d6 = d2 * d2 * d2
            d8 = d6 * d2
            c6t6 = cut6 + tmp6
            c8t8 = cut8 + tmp8
            dd = d_v * (1.0 / CUTOFF) - 1.0
            e6 = 1.0 / (d6 + tmp6) - 1.0 / c6t6 + (6.0 * cut6) / (c6t6 * c6t6) * dd
            e8 = 1.0 / (d8 + tmp8) - 1.0 / c8t8 + (8.0 * cut8) / (c8t8 * c8t8) * dd
            inside = d_v < CUTOFF
            e6 = jnp.where(inside, e6, 0.0)
            e8 = jnp.where(inside, e8, 0.0)
            e6 = (-0.5 * s6v) * c6 * e6
            e8 = (-0.5 * s8v) * c8 * e8
            ebuf[k, sl] = e6 + e8
            return 0
        lax.fori_loop(0, CB * 8, vblk, 0)

        for k in range(CB):
            pltpu.sync_copy(ebuf.at[k], out_sp.at[iib.at[b, k]], add=True)

    issue_linear(0, 0)
    wait_linear(0, 0)
    issue_indirect(0, 0)
    issue_linear(1, 1)

    def pair(g, _):
        for b in (0, 1):
            idx = g * 2 + b
            other = 1 - b

            @pl.when(idx + 1 < NCH2)
            def _():
                wait_linear(idx + 1, other)
                issue_indirect(idx + 1, other)

            wait_indirect(idx, b)
            compute(idx, b)

            @pl.when(idx + 2 < NCH2)
            def _():
                issue_linear(idx + 2, b)
        return 0
    lax.fori_loop(0, NCH2 // 2, pair, 0)

    plsc.subcore_barrier()
    pltpu.sync_copy(out_sp.at[ssl], b0)
    pltpu.sync_copy(b0, out2_h.at[pl.ds(c * NP + stripe, STRIPE)])


def _merge_body(a_ref, o_ref):
    o_ref[...] = a_ref[0] + a_ref[1]


def kernel(atomic_numbers, distances, idx_i, idx_j, c6ab, rcov, r2r4,
           s6, s8, a1, a2):
    f32 = jnp.float32
    i32 = jnp.int32
    zp = jnp.concatenate([atomic_numbers.astype(i32),
                          jnp.zeros((NP - N,), i32)])
    pad = E2 - E
    ii2 = jnp.concatenate([idx_i.astype(i32),
                           jnp.full((pad,), DEAD, i32)]).reshape(EROWS, 128)
    ij2 = jnp.concatenate([idx_j.astype(i32),
                           jnp.full((pad,), DEAD, i32)]).reshape(EROWS, 128)
    d2 = jnp.concatenate([distances,
                          jnp.full((pad,), 11.0, f32)]).reshape(EROWS, 128)
    rc96 = jnp.concatenate([rcov, jnp.ones((1,), f32)])
    sr96 = jnp.concatenate([jnp.sqrt(r2r4), jnp.ones((1,), f32)])
    c6t = c6ab.reshape(ZMAX * ZMAX, 75)
    par = jnp.stack([jnp.broadcast_to(s6, (16,)), jnp.broadcast_to(s8, (16,)),
                     jnp.broadcast_to(a1, (16,)), jnp.broadcast_to(a2, (16,))])

    pass1 = pl.kernel(
        _pass1_body,
        out_type=(jax.ShapeDtypeStruct((2 * NP,), f32),
                  jax.ShapeDtypeStruct((EROWS, 128), i32),
                  jax.ShapeDtypeStruct((EROWS, 128), f32)),
        mesh=_mesh,
        scratch_types=[
            pltpu.VMEM((NP,), i32),
            pltpu.VMEM((96,), f32),
            pltpu.VMEM((96,), f32),
            pltpu.VMEM((2, CA, 128), i32),
            pltpu.VMEM((2, CA, 128), i32),
            pltpu.VMEM((2, CA, 128), f32),
            pltpu.VMEM((2, CA, 128), f32),
            pltpu.VMEM((2, CA, 128), i32),
            pltpu.VMEM((2, CA, 128), f32),
            pltpu.VMEM((STRIPE,), f32),
            pltpu.SemaphoreType.DMA,
            pltpu.SemaphoreType.DMA,
            pltpu.SemaphoreType.DMA,
            pltpu.SemaphoreType.DMA,
            pltpu.VMEM_SHARED((NP,), f32),
        ],
        compiler_params=pltpu.CompilerParams(needs_layout_passes=False, use_tc_tiling_on_sc=False),
        name="d3_pass1",
    )
    nc2, p2, st2 = pass1(ii2, ij2, d2, zp, rc96, sr96)

    pass2 = pl.kernel(
        _pass2_body,
        out_type=jax.ShapeDtypeStruct((2 * NP,), f32),
        mesh=_mesh,
        scratch_types=[
            pltpu.VMEM((NP,), f32),
            pltpu.VMEM((STRIPE,), f32),
            pltpu.VMEM((STRIPE,), f32),
            pltpu.VMEM((2, CB, 128), i32),
            pltpu.VMEM((2, CB, 128), i32),
            pltpu.VMEM((2, CB, 128), f32),
            pltpu.VMEM((2, CB, 128), i32),
            pltpu.VMEM((2, CB, 128), f32),
            pltpu.VMEM((2, CB * 128, 75), f32),
            pltpu.VMEM((CB, 128), f32),
            pltpu.VMEM((4, 16), f32),
            pltpu.SemaphoreType.DMA,
            pltpu.SemaphoreType.DMA,
            pltpu.SemaphoreType.DMA,
            pltpu.SemaphoreType.DMA,
            pltpu.VMEM_SHARED((NP,), f32),
            pltpu.VMEM_SHARED((NP,), f32),
        ],
        compiler_params=pltpu.CompilerParams(needs_layout_passes=False, use_tc_tiling_on_sc=False),
        name="d3_pass2",
    )
    out2 = pass2(ii2, ij2, d2, p2, st2, nc2, c6t, par)

    merged = pl.pallas_call(
        _merge_body,
        out_shape=jax.ShapeDtypeStruct((EROWS // 32, 128), f32),
        grid=(EROWS // 32 // 56,),
        in_specs=[pl.BlockSpec((2, 56, 128), lambda i: (0, i, 0))],
        out_specs=pl.BlockSpec((56, 128), lambda i: (i, 0)),
    )(out2.reshape(2, EROWS // 32, 128))
    return merged.reshape(NP)[:N]
